# baseline (speedup 1.0000x reference)
import functools

import jax
import jax.numpy as jnp
from jax import lax
from jax.experimental import pallas as pl
from jax.experimental.pallas import tpu as pltpu
from jax.experimental.pallas import tpu_sc as plsc

L = 16
NW = 32
D = 64
V = 1000000
NBLK = V // 128
TAIL = V - NBLK * 128
CH = 128
NCH = 50
TPW = CH * NCH
PER = 200 * D


def _convert(tabT, tail128):
    mesh = plsc.VectorSubcoreMesh(core_axis_name="c", subcore_axis_name="s")

    @functools.partial(
        pl.kernel,
        mesh=mesh,
        compiler_params=pltpu.CompilerParams(needs_layout_passes=False),
        out_type=jax.ShapeDtypeStruct((V, 128), jnp.float32),
        scratch_types=[
            pltpu.VMEM((D, 128), jnp.float32),
            pltpu.VMEM((D, 128), jnp.float32),
            pltpu.VMEM((128, 128), jnp.float32),
            pltpu.VMEM((128, 128), jnp.float32),
            pltpu.SemaphoreType.DMA,
            pltpu.SemaphoreType.DMA,
            pltpu.SemaphoreType.DMA,
            pltpu.SemaphoreType.DMA,
        ],
    )
    def k(tabT_hbm, tail_hbm, out_hbm, blk0, blk1, tb0, tb1, rs0, rs1, ws0, ws1):
        wid = lax.axis_index("s") * 2 + lax.axis_index("c")
        blks = (blk0, blk1)
        tbs = (tb0, tb1)
        rsems = (rs0, rs1)
        wsems = (ws0, ws1)
        iota = lax.iota(jnp.int32, L)
        dvs = [u * L + iota for u in range(D // L)]
        nrounds = (NBLK + NW - 1) // NW

        def bid(i):
            return i * NW + wid

        def start_read(b, blk, sem):
            for tr in range(8):
                pltpu.async_copy(
                    tabT_hbm.at[pl.ds(tr * 8, 8), pl.ds(b * 128, 128)],
                    blk.at[pl.ds(tr * 8, 8)], sem)

        def wait_read(b, blk, sem):
            for tr in range(8):
                pltpu.make_async_copy(
                    tabT_hbm.at[pl.ds(tr * 8, 8), pl.ds(b * 128, 128)],
                    blk.at[pl.ds(tr * 8, 8)], sem).wait()

        @pl.when(bid(0) < NBLK)
        def _():
            start_read(bid(0), blk0, rs0)

        @pl.when(bid(1) < NBLK)
        def _():
            start_read(bid(1), blk1, rs1)

        def one_round(i, p):
            blk, tb = blks[p], tbs[p]
            b = bid(i)

            @pl.when(b < NBLK)
            def _():
                wait_read(b, blk, rsems[p])

                @pl.when(i >= 2)
                def _():
                    pltpu.make_async_copy(
                        tb, out_hbm.at[pl.ds(b * 128, 128)],
                        wsems[p]).wait()

                def trans8(t8, carry2):
                    vals = []
                    for rr in range(8):
                        tl = t8 * 8 + rr
                        tlv = jnp.broadcast_to(tl, (L,)).astype(jnp.int32)
                        for u in range(D // L):
                            vals.append((tl, u, plsc.load_gather(
                                blk, [dvs[u], tlv])))
                    for tl, u, v in vals:
                        tb[tl, pl.ds(u * L, L)] = v
                    return carry2

                lax.fori_loop(0, 16, trans8, 0)

                pltpu.async_copy(tb, out_hbm.at[pl.ds(b * 128, 128)],
                                 wsems[p])

                @pl.when(bid(i + 2) < NBLK)
                def _():
                    start_read(bid(i + 2), blk, rsems[p])

        def body(g, carry):
            for p in range(2):
                one_round(2 * g + p, p)
            return carry

        lax.fori_loop(0, nrounds // 2, body, 0)
        one_round(jnp.int32(nrounds - 1), 0)

        @pl.when(bid(nrounds - 1) < NBLK)
        def _():
            pltpu.make_async_copy(
                tb0, out_hbm.at[pl.ds(0, 128)], ws0).wait()

        @pl.when(bid(nrounds - 2) < NBLK)
        def _():
            pltpu.make_async_copy(
                tb1, out_hbm.at[pl.ds(0, 128)], ws1).wait()

        @pl.when(wid == 0)
        def _():
            pltpu.sync_copy(tail_hbm, tb0.at[pl.ds(0, TAIL)])
            pltpu.sync_copy(tb0.at[pl.ds(0, TAIL)],
                            out_hbm.at[pl.ds(NBLK * 128, TAIL)])

    return k(tabT, tail128)


def _gather(tok1d, tabP, pos1d):
    mesh = plsc.VectorSubcoreMesh(core_axis_name="c", subcore_axis_name="s")
    n_tok = tok1d.shape[0]

    @functools.partial(
        pl.kernel,
        mesh=mesh,
        compiler_params=pltpu.CompilerParams(needs_layout_passes=False),
        out_type=jax.ShapeDtypeStruct((n_tok, 128), jnp.float32),
        scratch_types=[
            pltpu.VMEM((TPW,), jnp.int32),
            pltpu.VMEM((PER,), jnp.float32),
            pltpu.VMEM((CH, 128), jnp.float32),
            pltpu.VMEM((CH, 128), jnp.float32),
            pltpu.SemaphoreType.DMA,
            pltpu.SemaphoreType.DMA,
            pltpu.SemaphoreType.DMA,
            pltpu.SemaphoreType.DMA,
        ],
    )
    def k(tok_hbm, tab_hbm, pos_hbm, out_hbm, tok_v, pos_v, buf0, buf1,
          gsem0, gsem1, osem0, osem1):
        wid = lax.axis_index("s") * 2 + lax.axis_index("c")
        bufs = (buf0, buf1)
        gsems = (gsem0, gsem1)
        osems = (osem0, osem1)

        pltpu.sync_copy(tok_hbm.at[pl.ds(wid * TPW, TPW)], tok_v)
        pltpu.sync_copy(pos_hbm, pos_v)

        def row0(c):
            return (wid * NCH + c) * CH

        pltpu.async_copy(tab_hbm.at[tok_v.at[pl.ds(0, CH)]], buf0, gsem0)
        pltpu.async_copy(tab_hbm.at[tok_v.at[pl.ds(CH, CH)]], buf1, gsem1)

        def super_body(g, carry):
            for p in range(2):
                c = 2 * g + p
                buf = bufs[p]
                pltpu.make_async_copy(
                    tab_hbm.at[tok_v.at[pl.ds(c * CH, CH)]], buf,
                    gsems[p]).wait()

                @pl.when(g >= 1)
                def _():
                    pltpu.make_async_copy(buf,
                                          out_hbm.at[pl.ds(row0(c), CH)],
                                          osems[p]).wait()

                po0 = lax.rem(c * (CH * D), PER)

                def add_row(r, po):
                    for u in range(D // L):
                        buf[r, pl.ds(u * L, L)] += pos_v[pl.ds(po + u * L, L)]
                    po = po + D
                    return lax.select(po >= PER, po - PER, po)

                lax.fori_loop(0, CH, add_row, po0)

                pltpu.async_copy(buf, out_hbm.at[pl.ds(row0(c), CH)],
                                 osems[p])

                @pl.when(g < (NCH // 2) - 1)
                def _():
                    pltpu.async_copy(
                        tab_hbm.at[tok_v.at[pl.ds((c + 2) * CH, CH)]], buf,
                        gsems[p])
            return carry

        lax.fori_loop(0, NCH // 2, super_body, 0)

        pltpu.make_async_copy(buf0, out_hbm.at[pl.ds(row0(NCH - 2), CH)],
                              osem0).wait()
        pltpu.make_async_copy(buf1, out_hbm.at[pl.ds(row0(NCH - 1), CH)],
                              osem1).wait()

    return k(tok1d, tabP, pos1d)


def kernel(tokens, token_table, position_embeddings):
    batch, n_token = tokens.shape
    tok1d = tokens.astype(jnp.int32).reshape(-1)
    pos1d = position_embeddings.reshape(-1)
    tail128 = jnp.pad(token_table[NBLK * 128:], ((0, 0), (0, 128 - D)))
    tabP = _convert(token_table.T, tail128)
    out = _gather(tok1d, tabP, pos1d)
    return out[:, :D].reshape(batch, n_token, D)


# speedup vs baseline: 1.5096x; 1.5096x over previous
import functools

import jax
import jax.numpy as jnp
from jax import lax
from jax.experimental import pallas as pl
from jax.experimental.pallas import tpu as pltpu
from jax.experimental.pallas import tpu_sc as plsc

L = 16
NW = 32
CH = 128
NCH = 50
TPW = CH * NCH
D = 64
PER = 200 * D


def _emb_call(tok1d, table, pos1d):
    mesh = plsc.VectorSubcoreMesh(core_axis_name="c", subcore_axis_name="s")
    n_tok = tok1d.shape[0]

    @functools.partial(
        pl.kernel,
        mesh=mesh,
        compiler_params=pltpu.CompilerParams(use_tc_tiling_on_sc=False),
        out_type=jax.ShapeDtypeStruct((n_tok, D), jnp.float32),
        scratch_types=[
            pltpu.VMEM((TPW,), jnp.int32),
            pltpu.VMEM((PER,), jnp.float32),
            pltpu.VMEM((CH, D), jnp.float32),
            pltpu.VMEM((CH, D), jnp.float32),
            pltpu.SemaphoreType.DMA,
            pltpu.SemaphoreType.DMA,
            pltpu.SemaphoreType.DMA,
            pltpu.SemaphoreType.DMA,
        ],
    )
    def k(tok_hbm, tab_hbm, pos_hbm, out_hbm, tok_v, pos_v, buf0, buf1,
          gsem0, gsem1, osem0, osem1):
        wid = lax.axis_index("s") * 2 + lax.axis_index("c")
        bufs = (buf0, buf1)
        gsems = (gsem0, gsem1)
        osems = (osem0, osem1)

        pltpu.sync_copy(tok_hbm.at[pl.ds(wid * TPW, TPW)], tok_v)
        pltpu.sync_copy(pos_hbm, pos_v)

        def row0(c):
            return (wid * NCH + c) * CH

        pltpu.async_copy(tab_hbm.at[tok_v.at[pl.ds(0, CH)]], buf0, gsem0)
        pltpu.async_copy(tab_hbm.at[tok_v.at[pl.ds(CH, CH)]], buf1, gsem1)

        def super_body(g, carry):
            for p in range(2):
                c = 2 * g + p
                buf = bufs[p]
                pltpu.make_async_copy(
                    tab_hbm.at[tok_v.at[pl.ds(c * CH, CH)]], buf,
                    gsems[p]).wait()
                @pl.when(g >= 1)
                def _():
                    pltpu.make_async_copy(buf,
                                          out_hbm.at[pl.ds(row0(c), CH)],
                                          osems[p]).wait()

                po0 = lax.rem(c * (CH * D), PER)

                def add_row(r, po):
                    for u in range(D // L):
                        buf[r, pl.ds(u * L, L)] += pos_v[pl.ds(po + u * L, L)]
                    po = po + D
                    return lax.select(po >= PER, po - PER, po)

                lax.fori_loop(0, CH, add_row, po0)

                pltpu.async_copy(buf, out_hbm.at[pl.ds(row0(c), CH)],
                                 osems[p])

                @pl.when(g < (NCH // 2) - 1)
                def _():
                    pltpu.async_copy(
                        tab_hbm.at[tok_v.at[pl.ds((c + 2) * CH, CH)]], buf,
                        gsems[p])
            return carry

        lax.fori_loop(0, NCH // 2, super_body, 0)

        pltpu.make_async_copy(buf0, out_hbm.at[pl.ds(row0(NCH - 2), CH)],
                              osem0).wait()
        pltpu.make_async_copy(buf1, out_hbm.at[pl.ds(row0(NCH - 1), CH)],
                              osem1).wait()

    return k(tok1d, table, pos1d)


def kernel(tokens, token_table, position_embeddings):
    batch, n_token = tokens.shape
    tok1d = tokens.astype(jnp.int32).reshape(-1)
    pos1d = position_embeddings.reshape(-1)
    out = _emb_call(tok1d, token_table, pos1d)
    return out.reshape(batch, n_token, D)
